# baseline (speedup 1.0000x reference)
import functools

import jax
import jax.numpy as jnp
from jax import lax
from jax.experimental import pallas as pl
from jax.experimental.pallas import tpu as pltpu
from jax.experimental.pallas import tpu_sc as plsc

DIM = 16
B = 16384
L = 20
NCTX = 2 * L
NVOCAB = 1000000

NC = 2
NS = 16
NW = NC * NS
BPW = B // NW
CB = 16
NCHUNK = BPW // CB
ROWS_PER_CHUNK = CB * NCTX
GATHER_ROWS = 128
NSTREAM = ROWS_PER_CHUNK // GATHER_ROWS


def _sc_body(ctx_idx_hbm, u_idx_hbm, u_tbl_hbm, v_tbl_hbm, out_hbm,
             idx_v, hi_v, u_idx_v, u_hi_v, u_rows, ctx_rows, acc_v, sem):
    wid = lax.axis_index("s") * NC + lax.axis_index("c")
    lane = jnp.arange(16, dtype=jnp.int32)

    def chunk_body(c, acc):
        pltpu.sync_copy(
            ctx_idx_hbm.at[pl.ds(wid * (BPW * NCTX) + c * ROWS_PER_CHUNK,
                                 ROWS_PER_CHUNK)],
            idx_v)
        pltpu.sync_copy(u_idx_hbm.at[pl.ds(wid * BPW + c * CB, CB)], u_idx_v)
        for k in range(ROWS_PER_CHUNK // 16):
            hi_v[pl.ds(k * 16, 16)] = idx_v[pl.ds(k * 16, 16)] >> 3
        u_hi_v[...] = u_idx_v[...] >> 3

        cps = [
            pltpu.async_copy(
                v_tbl_hbm.at[hi_v.at[pl.ds(k * GATHER_ROWS, GATHER_ROWS)]],
                ctx_rows.at[pl.ds(k * GATHER_ROWS, GATHER_ROWS)], sem)
            for k in range(NSTREAM)
        ]
        cps.append(pltpu.async_copy(u_tbl_hbm.at[u_hi_v], u_rows, sem))
        for cp in cps:
            cp.wait()

        usub16 = (u_idx_v[...] & 7) * 16
        u_lanes = [
            plsc.load_gather(u_rows, [lane, usub16 + d]) for d in range(DIM)
        ]
        base_rows = lane * NCTX

        def jbody(j, acc):
            rows = base_rows + j
            iv = plsc.load_gather(idx_v, [rows])
            sub16 = (iv & 7) * 16
            s = jnp.zeros((16,), jnp.float32)
            for d in range(DIM):
                cv = plsc.load_gather(ctx_rows, [rows, sub16 + d])
                s = s + cv * u_lanes[d]
            return acc + 1.0 / (1.0 + jnp.exp(s))

        return lax.fori_loop(0, NCTX, jbody, acc)

    acc = lax.fori_loop(0, NCHUNK, chunk_body, jnp.zeros((16,), jnp.float32))
    acc_v[...] = acc
    pltpu.sync_copy(acc_v, out_hbm.at[pl.ds(wid * 16, 16)])


@jax.jit
def _sc_call(ctx_idx, u_idx, u_tbl, v_tbl):
    mesh = plsc.VectorSubcoreMesh(core_axis_name="c", subcore_axis_name="s")
    f = functools.partial(
        pl.kernel,
        mesh=mesh,
        out_type=jax.ShapeDtypeStruct((NW * 16,), jnp.float32),
        scratch_types=[
            pltpu.VMEM((ROWS_PER_CHUNK,), jnp.int32),
            pltpu.VMEM((ROWS_PER_CHUNK,), jnp.int32),
            pltpu.VMEM((CB,), jnp.int32),
            pltpu.VMEM((CB,), jnp.int32),
            pltpu.VMEM((CB, 128), jnp.float32),
            pltpu.VMEM((ROWS_PER_CHUNK, 128), jnp.float32),
            pltpu.VMEM((16,), jnp.float32),
            pltpu.SemaphoreType.DMA,
        ],
        compiler_params=pltpu.CompilerParams(
            needs_layout_passes=False,
        ),
    )(_sc_body)
    return f(ctx_idx, u_idx, u_tbl, v_tbl)


def kernel(u, v, neg, u_emb_w, v_emb_w):
    u = u.astype(jnp.int32)
    ctx = jnp.concatenate([v.astype(jnp.int32), neg.astype(jnp.int32)],
                          axis=1)
    ctx_idx = ctx.reshape(B * NCTX)
    u_tbl = u_emb_w.reshape(NVOCAB // 8, 128)
    v_tbl = v_emb_w.reshape(NVOCAB // 8, 128)
    partial = _sc_call(ctx_idx, u, u_tbl, v_tbl)
    return jnp.sum(partial) / (B * NCTX)


# speedup vs baseline: 1.5744x; 1.5744x over previous
import functools

import jax
import jax.numpy as jnp
import numpy as np
from jax import lax
from jax.experimental import pallas as pl
from jax.experimental.pallas import tpu as pltpu
from jax.experimental.pallas import tpu_sc as plsc

DIM = 16
B = 16384
L = 20
NCTX = 2 * L
NVOCAB = 1000000

NC = 2
NS = 16
NW = NC * NS

TCOLS = NVOCAB // 128
SWEEP_TC = 4
SWEEP_V = SWEEP_TC * 128
COLS_PW = TCOLS // NW
NSWEEP = COLS_PW // SWEEP_TC
XCOLS = TCOLS - COLS_PW * NW
TAIL_V = NVOCAB - TCOLS * 128
SCR_ROWS = NVOCAB // 8

BPW = B // NW
CB = 64
NCHUNK = BPW // CB
GROUPS = CB // 16
ROWS_PER_CHUNK = CB * NCTX
GATHER_ROWS = 128
IDX_ROWS = ROWS_PER_CHUNK // GATHER_ROWS
U_IDX_ROWS = BPW // GATHER_ROWS

_LANE = np.arange(16, dtype=np.int32)


def _relayout_body(ut_hbm, vt_hbm, utail_hbm, vtail_hbm, scru_hbm, scrv_hbm,
                   in_v, out_v, tail_v):
    wid = lax.axis_index("s") * NC + lax.axis_index("c")

    lane16 = jnp.arange(16, dtype=jnp.int32) * 16
    rcs = [(lane16 + d) >> 7 for d in range(DIM)]
    ccs = [(lane16 + d) & 127 for d in range(DIM)]

    def do_sweep(src_hbm, dst_hbm, col):
        pltpu.sync_copy(src_hbm.at[:, pl.ds(col * 128, SWEEP_V)], in_v)
        for g in range(SWEEP_V // 16):
            base = jnp.full((16,), g * 2, jnp.int32)
            for d in range(DIM):
                vec = in_v[d, pl.ds(g * 16, 16)]
                plsc.store_scatter(out_v, [base + rcs[d], ccs[d]], vec)
        pltpu.sync_copy(out_v, dst_hbm.at[pl.ds(col * 16, SWEEP_V // 8)])

    def sweep_body(k, carry):
        col = wid * COLS_PW + k * SWEEP_TC
        do_sweep(ut_hbm, scru_hbm, col)
        do_sweep(vt_hbm, scrv_hbm, col)
        return carry

    lax.fori_loop(0, NSWEEP, sweep_body, 0)

    @pl.when(wid < XCOLS)
    def _():
        col = NW * COLS_PW + wid
        pltpu.sync_copy(ut_hbm.at[:, pl.ds(col * 128, 128)],
                        in_v.at[:, pl.ds(0, 128)])
        pltpu.sync_copy(vt_hbm.at[:, pl.ds(col * 128, 128)],
                        in_v.at[:, pl.ds(128, 128)])
        for t in range(2):
            for g in range(8):
                base = jnp.full((16,), g * 2, jnp.int32)
                for d in range(DIM):
                    vec = in_v[d, pl.ds(t * 128 + g * 16, 16)]
                    plsc.store_scatter(
                        out_v, [base + rcs[d] + t * 16, ccs[d]], vec)
        pltpu.sync_copy(out_v.at[pl.ds(0, 16)],
                        scru_hbm.at[pl.ds(col * 16, 16)])
        pltpu.sync_copy(out_v.at[pl.ds(16, 16)],
                        scrv_hbm.at[pl.ds(col * 16, 16)])

    @pl.when(wid == NW - 1)
    def _():
        pltpu.sync_copy(utail_hbm, tail_v)
        pltpu.sync_copy(tail_v, scru_hbm.at[pl.ds(SCR_ROWS - 8, 8)])
        pltpu.sync_copy(vtail_hbm, tail_v)
        pltpu.sync_copy(tail_v, scrv_hbm.at[pl.ds(SCR_ROWS - 8, 8)])


def _score_body(ctx_idx_hbm, u_idx_hbm, u_emb_hbm, v_emb_hbm, out_hbm,
                ctx_idx_v, u_idx_v, u_rows, ctx_rows, acc_v, sem):
    wid = lax.axis_index("s") * NC + lax.axis_index("c")
    lane = jnp.arange(16, dtype=jnp.int32)

    pltpu.sync_copy(u_idx_hbm.at[pl.ds(wid * BPW, BPW)], u_idx_v)
    cps = [
        pltpu.async_copy(u_emb_hbm.at[u_idx_v.at[pl.ds(i * GATHER_ROWS,
                                                       GATHER_ROWS)]],
                         u_rows.at[pl.ds(i * GATHER_ROWS, GATHER_ROWS)], sem)
        for i in range(U_IDX_ROWS)
    ]
    for cp in cps:
        cp.wait()

    acc = jnp.zeros((16,), jnp.float32)
    for c in range(NCHUNK):
        pltpu.sync_copy(
            ctx_idx_hbm.at[pl.ds(wid * (BPW * NCTX) + c * ROWS_PER_CHUNK,
                                 ROWS_PER_CHUNK)],
            ctx_idx_v)
        cps = [
            pltpu.async_copy(
                v_emb_hbm.at[ctx_idx_v.at[pl.ds(i * GATHER_ROWS,
                                                GATHER_ROWS)]],
                ctx_rows.at[pl.ds(i * GATHER_ROWS, GATHER_ROWS)], sem)
            for i in range(IDX_ROWS)
        ]
        for cp in cps:
            cp.wait()

        for g in range(GROUPS):
            off = c * CB + g * 16
            u_lanes = [
                plsc.load_gather(
                    u_rows, [lane + off, jnp.full((16,), d, jnp.int32)])
                for d in range(DIM)
            ]
            base_rows = g * 16 * NCTX + lane * NCTX

            def jbody(j, acc, base_rows=base_rows, u_lanes=u_lanes):
                rows = base_rows + j
                s = jnp.zeros((16,), jnp.float32)
                for d in range(DIM):
                    cv = plsc.load_gather(
                        ctx_rows, [rows, jnp.full((16,), d, jnp.int32)])
                    s = s + cv * u_lanes[d]
                return acc + 1.0 / (1.0 + jnp.exp(s))

            acc = lax.fori_loop(0, NCTX, jbody, acc)

    acc_v[...] = acc
    pltpu.sync_copy(acc_v, out_hbm.at[pl.ds(wid * 16, 16)])


@jax.jit
def _sc_call(ctx_idx, u_idx, ut, vt, utail, vtail):
    mesh = plsc.VectorSubcoreMesh(core_axis_name="c", subcore_axis_name="s")
    relayout = functools.partial(
        pl.kernel,
        mesh=mesh,
        out_type=(jax.ShapeDtypeStruct((SCR_ROWS, 128), jnp.float32),
                  jax.ShapeDtypeStruct((SCR_ROWS, 128), jnp.float32)),
        scratch_types=[
            pltpu.VMEM((16, SWEEP_V), jnp.float32),
            pltpu.VMEM((SWEEP_V // 8, 128), jnp.float32),
            pltpu.VMEM((8, 128), jnp.float32),
        ],
        compiler_params=pltpu.CompilerParams(
            needs_layout_passes=False,
        ),
    )(_relayout_body)
    scru, scrv = relayout(ut, vt, utail, vtail)

    score = functools.partial(
        pl.kernel,
        mesh=mesh,
        out_type=jax.ShapeDtypeStruct((NW * 16,), jnp.float32),
        scratch_types=[
            pltpu.VMEM((ROWS_PER_CHUNK,), jnp.int32),
            pltpu.VMEM((BPW,), jnp.int32),
            pltpu.VMEM((BPW, DIM), jnp.float32),
            pltpu.VMEM((ROWS_PER_CHUNK, DIM), jnp.float32),
            pltpu.VMEM((16,), jnp.float32),
            pltpu.SemaphoreType.DMA,
        ],
        compiler_params=pltpu.CompilerParams(
            needs_layout_passes=False,
            use_tc_tiling_on_sc=False,
        ),
    )(_score_body)
    return score(ctx_idx, u_idx,
                 scru.reshape(NVOCAB, DIM), scrv.reshape(NVOCAB, DIM))


def kernel(u, v, neg, u_emb_w, v_emb_w):
    u = u.astype(jnp.int32)
    ctx = jnp.concatenate([v.astype(jnp.int32), neg.astype(jnp.int32)],
                          axis=1)
    ctx_idx = ctx.reshape(B * NCTX)
    utail = u_emb_w[TCOLS * 128:].reshape(8, 128)
    vtail = v_emb_w[TCOLS * 128:].reshape(8, 128)
    partial = _sc_call(ctx_idx, u, u_emb_w.T, v_emb_w.T, utail, vtail)
    return jnp.sum(partial) / (B * NCTX)


# speedup vs baseline: 1.8188x; 1.1553x over previous
import functools

import jax
import jax.numpy as jnp
import numpy as np
from jax import lax
from jax.experimental import pallas as pl
from jax.experimental.pallas import tpu as pltpu
from jax.experimental.pallas import tpu_sc as plsc

DIM = 16
B = 16384
L = 20
NCTX = 2 * L
NVOCAB = 1000000

NC = 2
NS = 16
NW = NC * NS

TCOLS = NVOCAB // 128
SWEEP_TC = 4
SWEEP_V = SWEEP_TC * 128
COLS_PW = TCOLS // NW
NSWEEP = COLS_PW // SWEEP_TC
XCOLS = TCOLS - COLS_PW * NW
TAIL_V = NVOCAB - TCOLS * 128
SCR_ROWS = NVOCAB // 8

BPW = B // NW
CB = 64
NCHUNK = BPW // CB
GROUPS = CB // 16
ROWS_PER_CHUNK = CB * NCTX
GATHER_ROWS = 128
IDX_ROWS = ROWS_PER_CHUNK // GATHER_ROWS
U_IDX_ROWS = BPW // GATHER_ROWS

_LANE = np.arange(16, dtype=np.int32)


def _relayout_body(ut_hbm, vt_hbm, utail_hbm, vtail_hbm, scru_hbm, scrv_hbm,
                   in_u, in_w, out_u, out_w, tail_v,
                   sem_iu, sem_iv, sem_ou, sem_ov):
    wid = lax.axis_index("s") * NC + lax.axis_index("c")

    lane16 = jnp.arange(16, dtype=jnp.int32) * 16
    rcs = [(lane16 + d) >> 7 for d in range(DIM)]
    ccs = [(lane16 + d) & 127 for d in range(DIM)]

    def transpose_sweep(in_v, out_v):
        for g in range(SWEEP_V // 16):
            base = jnp.full((16,), g * 2, jnp.int32)
            for d in range(DIM):
                vec = in_v[d, pl.ds(g * 16, 16)]
                plsc.store_scatter(out_v, [base + rcs[d], ccs[d]], vec)

    def sweep_body(k, carry):
        col = wid * COLS_PW + k * SWEEP_TC
        cpu = pltpu.async_copy(
            ut_hbm.at[:, pl.ds(col * 128, SWEEP_V)], in_u, sem_iu)
        cpv = pltpu.async_copy(
            vt_hbm.at[:, pl.ds(col * 128, SWEEP_V)], in_w, sem_iv)
        cpu.wait()
        transpose_sweep(in_u, out_u)
        cpou = pltpu.async_copy(
            out_u, scru_hbm.at[pl.ds(col * 16, SWEEP_V // 8)], sem_ou)
        cpv.wait()
        transpose_sweep(in_w, out_w)
        cpov = pltpu.async_copy(
            out_w, scrv_hbm.at[pl.ds(col * 16, SWEEP_V // 8)], sem_ov)
        cpou.wait()
        cpov.wait()
        return carry

    lax.fori_loop(0, NSWEEP, sweep_body, 0)

    @pl.when(wid < XCOLS)
    def _():
        col = NW * COLS_PW + wid
        pltpu.sync_copy(ut_hbm.at[:, pl.ds(col * 128, 128)],
                        in_u.at[:, pl.ds(0, 128)])
        pltpu.sync_copy(vt_hbm.at[:, pl.ds(col * 128, 128)],
                        in_u.at[:, pl.ds(128, 128)])
        for t in range(2):
            for g in range(8):
                base = jnp.full((16,), g * 2, jnp.int32)
                for d in range(DIM):
                    vec = in_u[d, pl.ds(t * 128 + g * 16, 16)]
                    plsc.store_scatter(
                        out_u, [base + rcs[d] + t * 16, ccs[d]], vec)
        pltpu.sync_copy(out_u.at[pl.ds(0, 16)],
                        scru_hbm.at[pl.ds(col * 16, 16)])
        pltpu.sync_copy(out_u.at[pl.ds(16, 16)],
                        scrv_hbm.at[pl.ds(col * 16, 16)])

    @pl.when(wid == NW - 1)
    def _():
        pltpu.sync_copy(utail_hbm, tail_v)
        pltpu.sync_copy(tail_v, scru_hbm.at[pl.ds(SCR_ROWS - 8, 8)])
        pltpu.sync_copy(vtail_hbm, tail_v)
        pltpu.sync_copy(tail_v, scrv_hbm.at[pl.ds(SCR_ROWS - 8, 8)])


def _score_body(ctx_idx_hbm, u_idx_hbm, u_emb_hbm, v_emb_hbm, out_hbm,
                ctx_idx_a, ctx_idx_b, u_idx_v, u_rows, ctx_rows_a,
                ctx_rows_b, acc_v, sem_u, sem_a, sem_b):
    wid = lax.axis_index("s") * NC + lax.axis_index("c")
    lane = jnp.arange(16, dtype=jnp.int32)
    idx_bufs = (ctx_idx_a, ctx_idx_b)
    row_bufs = (ctx_rows_a, ctx_rows_b)
    sems = (sem_a, sem_b)

    pltpu.sync_copy(u_idx_hbm.at[pl.ds(wid * BPW, BPW)], u_idx_v)
    ucps = [
        pltpu.async_copy(u_emb_hbm.at[u_idx_v.at[pl.ds(i * GATHER_ROWS,
                                                       GATHER_ROWS)]],
                         u_rows.at[pl.ds(i * GATHER_ROWS, GATHER_ROWS)],
                         sem_u)
        for i in range(U_IDX_ROWS)
    ]

    def fire_chunk(c):
        buf = c % 2
        pltpu.sync_copy(
            ctx_idx_hbm.at[pl.ds(wid * (BPW * NCTX) + c * ROWS_PER_CHUNK,
                                 ROWS_PER_CHUNK)],
            idx_bufs[buf])
        return [
            pltpu.async_copy(
                v_emb_hbm.at[idx_bufs[buf].at[pl.ds(i * GATHER_ROWS,
                                                    GATHER_ROWS)]],
                row_bufs[buf].at[pl.ds(i * GATHER_ROWS, GATHER_ROWS)],
                sems[buf])
            for i in range(IDX_ROWS)
        ]

    pending = fire_chunk(0)
    for cp in ucps:
        cp.wait()

    acc = jnp.zeros((16,), jnp.float32)
    for c in range(NCHUNK):
        nxt = fire_chunk(c + 1) if c + 1 < NCHUNK else []
        for cp in pending:
            cp.wait()
        pending = nxt
        ctx_rows = row_bufs[c % 2]

        for g in range(GROUPS):
            off = c * CB + g * 16
            u_lanes = [
                plsc.load_gather(
                    u_rows, [lane + off, jnp.full((16,), d, jnp.int32)])
                for d in range(DIM)
            ]
            base_rows = g * 16 * NCTX + lane * NCTX

            def jbody(j, acc, base_rows=base_rows, u_lanes=u_lanes,
                      ctx_rows=ctx_rows):
                rows = base_rows + j
                s = jnp.zeros((16,), jnp.float32)
                for d in range(DIM):
                    cv = plsc.load_gather(
                        ctx_rows, [rows, jnp.full((16,), d, jnp.int32)])
                    s = s + cv * u_lanes[d]
                return acc + 1.0 / (1.0 + jnp.exp(s))

            acc = lax.fori_loop(0, NCTX, jbody, acc)

    acc_v[...] = acc
    pltpu.sync_copy(acc_v, out_hbm.at[pl.ds(wid * 16, 16)])


@jax.jit
def _sc_call(ctx_idx, u_idx, ut, vt, utail, vtail):
    mesh = plsc.VectorSubcoreMesh(core_axis_name="c", subcore_axis_name="s")
    relayout = functools.partial(
        pl.kernel,
        mesh=mesh,
        out_type=(jax.ShapeDtypeStruct((SCR_ROWS, 128), jnp.float32),
                  jax.ShapeDtypeStruct((SCR_ROWS, 128), jnp.float32)),
        scratch_types=[
            pltpu.VMEM((16, SWEEP_V), jnp.float32),
            pltpu.VMEM((16, SWEEP_V), jnp.float32),
            pltpu.VMEM((SWEEP_V // 8, 128), jnp.float32),
            pltpu.VMEM((SWEEP_V // 8, 128), jnp.float32),
            pltpu.VMEM((8, 128), jnp.float32),
            pltpu.SemaphoreType.DMA,
            pltpu.SemaphoreType.DMA,
            pltpu.SemaphoreType.DMA,
            pltpu.SemaphoreType.DMA,
        ],
        compiler_params=pltpu.CompilerParams(
            needs_layout_passes=False,
        ),
    )(_relayout_body)
    scru, scrv = relayout(ut, vt, utail, vtail)

    score = functools.partial(
        pl.kernel,
        mesh=mesh,
        out_type=jax.ShapeDtypeStruct((NW * 16,), jnp.float32),
        scratch_types=[
            pltpu.VMEM((ROWS_PER_CHUNK,), jnp.int32),
            pltpu.VMEM((ROWS_PER_CHUNK,), jnp.int32),
            pltpu.VMEM((BPW,), jnp.int32),
            pltpu.VMEM((BPW, DIM), jnp.float32),
            pltpu.VMEM((ROWS_PER_CHUNK, DIM), jnp.float32),
            pltpu.VMEM((ROWS_PER_CHUNK, DIM), jnp.float32),
            pltpu.VMEM((16,), jnp.float32),
            pltpu.SemaphoreType.DMA,
            pltpu.SemaphoreType.DMA,
            pltpu.SemaphoreType.DMA,
        ],
        compiler_params=pltpu.CompilerParams(
            needs_layout_passes=False,
            use_tc_tiling_on_sc=False,
        ),
    )(_score_body)
    return score(ctx_idx, u_idx,
                 scru.reshape(NVOCAB, DIM), scrv.reshape(NVOCAB, DIM))


def kernel(u, v, neg, u_emb_w, v_emb_w):
    u = u.astype(jnp.int32)
    ctx = jnp.concatenate([v.astype(jnp.int32), neg.astype(jnp.int32)],
                          axis=1)
    ctx_idx = ctx.reshape(B * NCTX)
    utail = u_emb_w[TCOLS * 128:].reshape(8, 128)
    vtail = v_emb_w[TCOLS * 128:].reshape(8, 128)
    partial = _sc_call(ctx_idx, u, u_emb_w.T, v_emb_w.T, utail, vtail)
    return jnp.sum(partial) / (B * NCTX)
